# baseline (speedup 1.0000x reference)
import functools

import jax
import jax.numpy as jnp
from jax import lax
from jax.experimental import pallas as pl
from jax.experimental.pallas import tpu as pltpu
from jax.experimental.pallas import tpu_sc as plsc

_NC = 2
_NS = 16
_NW = _NC * _NS
_L = 128
_NBUF = 8
_SS = 8


def _sc_gather(tt, idxs):
    D, V = tt.shape
    B = idxs.shape[0]
    perw = B // _NW
    mesh = plsc.VectorSubcoreMesh(core_axis_name="c", subcore_axis_name="s")

    @functools.partial(
        pl.kernel,
        mesh=mesh,
        out_type=jax.ShapeDtypeStruct((_NW, D, perw), jnp.float32),
        scratch_types=[
            pltpu.VMEM((perw + 16,), jnp.int32),
            pltpu.VMEM((_NBUF, D, _L), jnp.float32),
            pltpu.VMEM((D, perw), jnp.float32),
            pltpu.SemaphoreType.DMA((_NBUF,)),
        ],
        compiler_params=pltpu.CompilerParams(needs_layout_passes=False),
    )
    def gather_kernel(tt_hbm, idx_hbm, out_hbm, idx_v, colbuf, outbuf, sems):
        wid = lax.axis_index("s") * _NC + lax.axis_index("c")
        base = wid * perw

        @pl.when(wid < _NW)
        def _active():
            pltpu.sync_copy(idx_hbm.at[pl.ds(base, perw)], idx_v.at[pl.ds(0, perw)])

            def idx_at(j):
                return idx_v[pl.ds(j, 16)][0]

            def start_fetch(j):
                q = pl.multiple_of((idx_at(j) // _L) * _L, _L)
                pltpu.make_async_copy(
                    tt_hbm.at[:, pl.ds(q, _L)], colbuf.at[j % _NBUF], sems.at[j % _NBUF]
                ).start()

            for j in range(_NBUF):
                start_fetch(j)

            iota16 = lax.iota(jnp.int32, 16)

            def body(j, carry):
                pltpu.make_async_copy(
                    tt_hbm.at[:, pl.ds(0, _L)], colbuf.at[j % _NBUF], sems.at[j % _NBUF]
                ).wait()
                rv = jnp.full((16,), idx_at(j) % _L, jnp.int32)
                jv = jnp.full((16,), j, jnp.int32)
                tile = colbuf.at[j % _NBUF]
                for c in range(D // 16):
                    dv = iota16 + 16 * c
                    vals = plsc.load_gather(tile, [dv, rv])
                    plsc.store_scatter(outbuf, [dv, jv], vals)

                @pl.when(j + _NBUF < perw)
                def _():
                    start_fetch(j + _NBUF)

                return carry

            lax.fori_loop(0, perw, body, 0, unroll=False)
            pltpu.sync_copy(outbuf, out_hbm.at[wid])

    return gather_kernel(tt, idxs)


def _make_add_body(aliased, nw, perw):
    def body(*refs):
        if aliased:
            x_ref, e3_ref, _prev, o_ref, es_ref = refs
        else:
            x_ref, e3_ref, o_ref, es_ref = refs

        @pl.when(pl.program_id(0) == 0)
        def _():
            for w in range(nw):
                es_ref[:, perw * w:perw * (w + 1)] = e3_ref[w]

        o_ref[...] = x_ref[...] + es_ref[...]

    return body


_CHUNKS = ((0, 1024), (1024, 1024), (2048, 2048))


def kernel(x, idxs, embedding_weight):
    B, S, D = x.shape
    xv = jnp.transpose(x, (1, 2, 0))
    tt = embedding_weight.T

    embs = [_sc_gather(tt, idxs[off:off + bc]) for off, bc in _CHUNKS]

    out = None
    for c, (off, bc) in enumerate(_CHUNKS):
        perw = bc // _NW
        cblk = off // bc
        in_specs = [
            pl.BlockSpec((_SS, D, bc), lambda i, cblk=cblk: (i, 0, cblk)),
            pl.BlockSpec((_NW, D, perw), lambda i: (0, 0, 0)),
        ]
        args = [xv, embs[c]]
        if out is None:
            io_alias = {}
        else:
            in_specs.append(pl.BlockSpec(memory_space=pl.ANY))
            args.append(out)
            io_alias = {2: 0}
        out = pl.pallas_call(
            _make_add_body(out is not None, _NW, perw),
            grid=(S // _SS,),
            in_specs=in_specs,
            out_specs=pl.BlockSpec((_SS, D, bc), lambda i, cblk=cblk: (i, 0, cblk)),
            out_shape=jax.ShapeDtypeStruct((S, D, B), jnp.float32),
            input_output_aliases=io_alias,
            scratch_shapes=[pltpu.VMEM((D, bc), jnp.float32)],
        )(*args)
    return jnp.transpose(out, (2, 0, 1))


# speedup vs baseline: 1.0195x; 1.0195x over previous
import functools

import jax
import jax.numpy as jnp
from jax import lax
from jax.experimental import pallas as pl
from jax.experimental.pallas import tpu as pltpu
from jax.experimental.pallas import tpu_sc as plsc

_NC = 2
_NS = 16
_NW = _NC * _NS
_L = 128
_NBUF = 12
_SS = 10


def _sc_gather(tt, idxs):
    D, V = tt.shape
    B = idxs.shape[0]
    perw = B // _NW
    mesh = plsc.VectorSubcoreMesh(core_axis_name="c", subcore_axis_name="s")

    @functools.partial(
        pl.kernel,
        mesh=mesh,
        out_type=jax.ShapeDtypeStruct((_NW, D, perw), jnp.float32),
        scratch_types=[
            pltpu.VMEM((perw + 16,), jnp.int32),
            pltpu.VMEM((_NBUF, D, _L), jnp.float32),
            pltpu.VMEM((D, perw), jnp.float32),
            pltpu.SemaphoreType.DMA((_NBUF,)),
        ],
        compiler_params=pltpu.CompilerParams(needs_layout_passes=False),
    )
    def gather_kernel(tt_hbm, idx_hbm, out_hbm, idx_v, colbuf, outbuf, sems):
        wid = lax.axis_index("s") * _NC + lax.axis_index("c")
        base = wid * perw

        @pl.when(wid < _NW)
        def _active():
            pltpu.sync_copy(idx_hbm.at[pl.ds(base, perw)], idx_v.at[pl.ds(0, perw)])

            def idx_at(j):
                return idx_v[pl.ds(j, 16)][0]

            def start_fetch(j):
                q = pl.multiple_of((idx_at(j) // _L) * _L, _L)
                pltpu.make_async_copy(
                    tt_hbm.at[:, pl.ds(q, _L)], colbuf.at[j % _NBUF], sems.at[j % _NBUF]
                ).start()

            for j in range(_NBUF):
                start_fetch(j)

            iota16 = lax.iota(jnp.int32, 16)

            def body(j, carry):
                pltpu.make_async_copy(
                    tt_hbm.at[:, pl.ds(0, _L)], colbuf.at[j % _NBUF], sems.at[j % _NBUF]
                ).wait()
                rv = jnp.full((16,), idx_at(j) % _L, jnp.int32)
                jv = jnp.full((16,), j, jnp.int32)
                tile = colbuf.at[j % _NBUF]
                for c in range(D // 16):
                    dv = iota16 + 16 * c
                    vals = plsc.load_gather(tile, [dv, rv])
                    plsc.store_scatter(outbuf, [dv, jv], vals)

                @pl.when(j + _NBUF < perw)
                def _():
                    start_fetch(j + _NBUF)

                return carry

            lax.fori_loop(0, perw, body, 0, unroll=False)
            pltpu.sync_copy(outbuf, out_hbm.at[wid])

    return gather_kernel(tt, idxs)


def _make_add_body(aliased, nw, perw):
    def body(*refs):
        if aliased:
            x_ref, e3_ref, _prev, o_ref, es_ref = refs
        else:
            x_ref, e3_ref, o_ref, es_ref = refs

        @pl.when(pl.program_id(0) == 0)
        def _():
            for w in range(nw):
                es_ref[:, perw * w:perw * (w + 1)] = e3_ref[w]

        o_ref[...] = x_ref[...] + es_ref[...]

    return body


_CHUNKS = ((0, 2048), (2048, 2048))


def kernel(x, idxs, embedding_weight):
    B, S, D = x.shape
    xv = jnp.transpose(x, (1, 2, 0))
    tt = embedding_weight.T

    embs = [_sc_gather(tt, idxs[off:off + bc]) for off, bc in _CHUNKS]

    out = None
    for c, (off, bc) in enumerate(_CHUNKS):
        perw = bc // _NW
        cblk = off // bc
        in_specs = [
            pl.BlockSpec((_SS, D, bc), lambda i, cblk=cblk: (i, 0, cblk)),
            pl.BlockSpec((_NW, D, perw), lambda i: (0, 0, 0)),
        ]
        args = [xv, embs[c]]
        if out is None:
            io_alias = {}
        else:
            in_specs.append(pl.BlockSpec(memory_space=pl.ANY))
            args.append(out)
            io_alias = {2: 0}
        out = pl.pallas_call(
            _make_add_body(out is not None, _NW, perw),
            grid=(S // _SS,),
            in_specs=in_specs,
            out_specs=pl.BlockSpec((_SS, D, bc), lambda i, cblk=cblk: (i, 0, cblk)),
            out_shape=jax.ShapeDtypeStruct((S, D, B), jnp.float32),
            input_output_aliases=io_alias,
            scratch_shapes=[pltpu.VMEM((D, bc), jnp.float32)],
        )(*args)
    return jnp.transpose(out, (2, 0, 1))
